# baseline (speedup 1.0000x reference)
import functools

import jax
import jax.numpy as jnp
from jax import lax
from jax.experimental import pallas as pl
from jax.experimental.pallas import tpu as pltpu
from jax.experimental.pallas import tpu_sc as plsc

_DIM = 64
_L = 16
_MARGIN = 1.0


def _transe_body(C, n_chunks,
                 rph, rpr, rpt, rnh, rnr, rnt, ent, rel, out,
                 iph, ipr, ipt, inh, inr, int_,
                 vph, vpr, vpt, vnh, vnr, vnt, sbuf, acc_ref, sem):
    nc = 2
    wid = lax.axis_index("s") * nc + lax.axis_index("c")
    base = wid * (C * n_chunks)
    lane = lax.iota(jnp.int32, _L)

    acc_ref[...] = jnp.zeros((_L,), jnp.float32)
    for c in range(n_chunks):
        off = base + c * C
        pltpu.sync_copy(rph.at[pl.ds(off, C)], iph)
        pltpu.sync_copy(rpr.at[pl.ds(off, C)], ipr)
        pltpu.sync_copy(rpt.at[pl.ds(off, C)], ipt)
        pltpu.sync_copy(rnh.at[pl.ds(off, C)], inh)
        pltpu.sync_copy(rnr.at[pl.ds(off, C)], inr)
        pltpu.sync_copy(rnt.at[pl.ds(off, C)], int_)
        d0 = pltpu.async_copy(ent.at[iph], vph, sem)
        d1 = pltpu.async_copy(rel.at[ipr], vpr, sem)
        d2 = pltpu.async_copy(ent.at[ipt], vpt, sem)
        d3 = pltpu.async_copy(ent.at[inh], vnh, sem)
        d4 = pltpu.async_copy(rel.at[inr], vnr, sem)
        d5 = pltpu.async_copy(ent.at[int_], vnt, sem)
        d0.wait(); d1.wait(); d2.wait(); d3.wait(); d4.wait(); d5.wait()

        def group(g, carry):
            for j in range(_L):
                i = g * _L + j
                s = None
                for k in range(_DIM // _L):
                    sl = pl.ds(k * _L, _L)
                    p = jnp.abs(vph[i, sl] + vpr[i, sl] - vpt[i, sl])
                    n = jnp.abs(vnh[i, sl] + vnr[i, sl] - vnt[i, sl])
                    d = p - n
                    s = d if s is None else s + d
                plsc.store_scatter(sbuf, [lane * _L + j], s)
            tot = None
            for k in range(_L):
                v = sbuf[pl.ds(k * _L, _L)]
                tot = v if tot is None else tot + v
            hinge = jnp.maximum(tot + _MARGIN, 0.0)
            acc_ref[...] = acc_ref[...] + hinge
            return carry

        lax.fori_loop(0, C // _L, group, jnp.int32(0))

    pltpu.sync_copy(acc_ref, out.at[wid])


def _transe_sc(rph, rpr, rpt, rnh, rnr, rnt, ent, rel):
    B = rph.shape[0]
    nw = 32
    C = 128
    n_chunks = B // (nw * C)
    mesh = plsc.VectorSubcoreMesh(core_axis_name="c", subcore_axis_name="s")
    idx_t = pltpu.VMEM((C,), jnp.int32)
    row_t = pltpu.VMEM((C, _DIM), jnp.float32)
    kern = pl.kernel(
        functools.partial(_transe_body, C, n_chunks),
        mesh=mesh,
        compiler_params=pltpu.CompilerParams(needs_layout_passes=False,
                                             use_tc_tiling_on_sc=False),
        out_type=jax.ShapeDtypeStruct((nw, _L), jnp.float32),
        scratch_types=[idx_t] * 6 + [row_t] * 6 + [
            pltpu.VMEM((_L * _L,), jnp.float32),
            pltpu.VMEM((_L,), jnp.float32),
            pltpu.SemaphoreType.DMA,
        ],
    )
    return kern(rph, rpr, rpt, rnh, rnr, rnt, ent, rel)


def kernel(r_p_h, r_p_r, r_p_t, r_n_h, r_n_r, r_n_t, ent_embed, rel_embed):
    B = r_p_h.shape[0]
    partials = _transe_sc(
        r_p_h.astype(jnp.int32), r_p_r.astype(jnp.int32),
        r_p_t.astype(jnp.int32), r_n_h.astype(jnp.int32),
        r_n_r.astype(jnp.int32), r_n_t.astype(jnp.int32),
        ent_embed, rel_embed)
    return jnp.sum(partials) * jnp.float32(1.0 / B)
